# baseline (speedup 1.0000x reference)
import jax
import jax.numpy as jnp
from jax.experimental import pallas as pl
from jax.experimental.pallas import tpu as pltpu

_B, _S, _HID = 1, 2048, 2048
_NH, _NKV, _HD = 16, 8, 128
_ROPE_THETA = 10000.0
_SINK, _LOCAL = 64, 256
_NRET = 8
_REP = _NH // _NKV
_QKV_W = (_NH + 2 * _NKV) * _HD
_BQ = 256
_BK = 256
_NQ = _S // _BQ
_NK = _S // _BK
_SCALE = _HD ** -0.5
_NEG = -1e30


def _qkv_proj_kernel(x_ref, w_ref, o_ref):
    o_ref[...] = jnp.dot(x_ref[...], w_ref[...],
                         preferred_element_type=jnp.float32).astype(jnp.bfloat16)


def _out_proj_kernel(x_ref, w_ref, o_ref):
    o_ref[...] = jnp.dot(x_ref[...], w_ref[...],
                         preferred_element_type=jnp.float32)


def _kb_eff(h, qb, kb):
    ret = jnp.minimum(kb, qb)
    stream = jnp.where(kb == 0, 0,
                       jnp.where(kb == 1, jnp.maximum(qb - 1, 0), qb))
    return jnp.where(h < _NRET, ret, stream)


def _rope(x, cos, sin):
    x1 = x[:, : _HD // 2]
    x2 = x[:, _HD // 2:]
    rot = jnp.concatenate([-x2, x1], axis=-1)
    return x * cos + rot * sin


def _attn_kernel(q_ref, k_ref, v_ref, cq_ref, sq_ref, ck_ref, sk_ref,
                 o_ref, acc_ref, m_ref, l_ref):
    h = pl.program_id(0)
    qb = pl.program_id(1)
    kb = pl.program_id(2)

    @pl.when(kb == 0)
    def _init():
        m_ref[...] = jnp.full_like(m_ref, _NEG)
        l_ref[...] = jnp.zeros_like(l_ref)
        acc_ref[...] = jnp.zeros_like(acc_ref)

    valid_ret = kb <= qb
    valid_stream = ((kb == 0)
                    | ((kb == 1) & (qb >= 2))
                    | ((kb == 2) & (qb >= 1)))
    valid = jnp.where(h < _NRET, valid_ret, valid_stream)

    @pl.when(valid)
    def _update():
        kbe = _kb_eff(h, qb, kb)
        q = _rope(q_ref[...].astype(jnp.float32),
                  cq_ref[...], sq_ref[...]).astype(jnp.bfloat16)
        k = _rope(k_ref[...].astype(jnp.float32),
                  ck_ref[...], sk_ref[...]).astype(jnp.bfloat16)
        s = jax.lax.dot_general(
            q, k, (((1,), (1,)), ((), ())),
            preferred_element_type=jnp.float32) * _SCALE

        qpos = qb * _BQ + jax.lax.broadcasted_iota(jnp.int32, (_BQ, _BK), 0)
        kpos = kbe * _BK + jax.lax.broadcasted_iota(jnp.int32, (_BQ, _BK), 1)
        causal = kpos <= qpos
        window = (kpos < _SINK) | ((qpos - kpos) < _LOCAL)
        mask = causal & (window | (h < _NRET))
        s = jnp.where(mask, s, _NEG)

        m_prev = m_ref[...]
        curr = jnp.max(s, axis=1, keepdims=True)
        m_new = jnp.maximum(m_prev, curr)
        alpha = jnp.exp(m_prev - m_new)
        p = jnp.exp(s - m_new[:, 0:1])
        l_ref[...] = alpha * l_ref[...] + jnp.sum(p, axis=1, keepdims=True)
        acc_ref[...] = acc_ref[...] * alpha + jnp.dot(
            p.astype(jnp.bfloat16), v_ref[...],
            preferred_element_type=jnp.float32)
        m_ref[...] = m_new

    @pl.when(kb == _NK - 1)
    def _finish():
        o_ref[...] = (acc_ref[...] / l_ref[...]).astype(jnp.bfloat16)


def kernel(hidden_states, wqkv, wo):
    hs = hidden_states.reshape(_S, _HID).astype(jnp.bfloat16)
    wq = wqkv.astype(jnp.bfloat16)
    wo_b = wo.astype(jnp.bfloat16)

    qkv = pl.pallas_call(
        _qkv_proj_kernel,
        grid=(_NQ,),
        in_specs=[
            pl.BlockSpec((_BQ, _HID), lambda i: (i, 0)),
            pl.BlockSpec((_HID, _QKV_W), lambda i: (0, 0)),
        ],
        out_specs=pl.BlockSpec((_BQ, _QKV_W), lambda i: (i, 0)),
        out_shape=jax.ShapeDtypeStruct((_S, _QKV_W), jnp.bfloat16),
    )(hs, wq)

    half = _HD // 2
    inv_freq = 1.0 / (_ROPE_THETA ** (
        jnp.arange(0, half, dtype=jnp.float32) / half))
    pos = jnp.arange(_S, dtype=jnp.float32)
    angles = pos[:, None] * inv_freq[None, :]
    cos = jnp.cos(angles)
    sin = jnp.sin(angles)
    ctab = jnp.concatenate([cos, cos], axis=-1)
    stab = jnp.concatenate([sin, sin], axis=-1)

    attn = pl.pallas_call(
        _attn_kernel,
        grid=(_NH, _NQ, _NK),
        in_specs=[
            pl.BlockSpec((_BQ, _HD), lambda h, qb, kb: (qb, h)),
            pl.BlockSpec((_BK, _HD),
                         lambda h, qb, kb: (_kb_eff(h, qb, kb),
                                            _NH + h // _REP)),
            pl.BlockSpec((_BK, _HD),
                         lambda h, qb, kb: (_kb_eff(h, qb, kb),
                                            _NH + _NKV + h // _REP)),
            pl.BlockSpec((_BQ, _HD), lambda h, qb, kb: (qb, 0)),
            pl.BlockSpec((_BQ, _HD), lambda h, qb, kb: (qb, 0)),
            pl.BlockSpec((_BK, _HD),
                         lambda h, qb, kb: (_kb_eff(h, qb, kb), 0)),
            pl.BlockSpec((_BK, _HD),
                         lambda h, qb, kb: (_kb_eff(h, qb, kb), 0)),
        ],
        out_specs=pl.BlockSpec((_BQ, _HD), lambda h, qb, kb: (qb, h)),
        out_shape=jax.ShapeDtypeStruct((_S, _NH * _HD), jnp.bfloat16),
        scratch_shapes=[
            pltpu.VMEM((_BQ, _HD), jnp.float32),
            pltpu.VMEM((_BQ, _HD), jnp.float32),
            pltpu.VMEM((_BQ, _HD), jnp.float32),
        ],
    )(qkv, qkv, qkv, ctab, stab, ctab, stab)

    out = pl.pallas_call(
        _out_proj_kernel,
        grid=(_NQ,),
        in_specs=[
            pl.BlockSpec((_BQ, _NH * _HD), lambda i: (i, 0)),
            pl.BlockSpec((_NH * _HD, _HID), lambda i: (0, 0)),
        ],
        out_specs=pl.BlockSpec((_BQ, _HID), lambda i: (i, 0)),
        out_shape=jax.ShapeDtypeStruct((_S, _HID), jnp.float32),
    )(attn, wo_b)

    return out.reshape(_B, _S, _HID)


# speedup vs baseline: 2.1526x; 2.1526x over previous
import jax
import jax.numpy as jnp
from jax.experimental import pallas as pl
from jax.experimental.pallas import tpu as pltpu

_B, _S, _HID = 1, 2048, 2048
_NH, _NKV, _HD = 16, 8, 128
_ROPE_THETA = 10000.0
_SINK, _LOCAL = 64, 256
_NRET = 8
_REP = _NH // _NKV
_QKV_W = (_NH + 2 * _NKV) * _HD
_QK_W = (_NH + _NKV) * _HD
_NQK = _NH + _NKV
_BQ = 256
_NQ = _S // _BQ
_SCALE = _HD ** -0.5
_NEG = -1e30


def _qkv_rope_kernel(x_ref, w_ref, ct_ref, st_ref, o_ref):
    acc = jnp.dot(x_ref[...], w_ref[...], preferred_element_type=jnp.float32)
    qk = acc[:, :_QK_W]
    pieces = []
    for c in range(_NQK):
        x1 = qk[:, c * _HD: c * _HD + _HD // 2]
        x2 = qk[:, c * _HD + _HD // 2: (c + 1) * _HD]
        pieces += [-x2, x1]
    rot = jnp.concatenate(pieces, axis=1)
    roped = qk * ct_ref[...] + rot * st_ref[...]
    o_ref[...] = jnp.concatenate(
        [roped, acc[:, _QK_W:]], axis=1).astype(jnp.bfloat16)


def _out_proj_kernel(x_ref, w_ref, o_ref):
    o_ref[...] = jnp.dot(x_ref[...], w_ref[...],
                         preferred_element_type=jnp.float32)


def _attn_kernel(q_ref, k_ref, v_ref, o_ref):
    h = pl.program_id(0)
    qb = pl.program_id(1)
    qpos = qb * _BQ + jax.lax.broadcasted_iota(jnp.int32, (_BQ, 1), 0)

    @pl.when(h < _NRET)
    def _retrieval():
        q = q_ref[...]
        s = jax.lax.dot_general(
            q, k_ref[...], (((1,), (1,)), ((), ())),
            preferred_element_type=jnp.float32) * _SCALE
        kpos = jax.lax.broadcasted_iota(jnp.int32, (_BQ, _S), 1)
        s = jnp.where(kpos <= qpos, s, _NEG)
        m = jnp.max(s, axis=1, keepdims=True)
        p = jnp.exp(s - m)
        l = jnp.sum(p, axis=1, keepdims=True)
        o = jnp.dot(p.astype(jnp.bfloat16), v_ref[...],
                    preferred_element_type=jnp.float32)
        o_ref[...] = (o / l).astype(jnp.bfloat16)

    @pl.when(h >= _NRET)
    def _streaming():
        q = q_ref[...]
        wstart = jnp.maximum(qb - 1, 0) * _BQ
        kw = k_ref[pl.ds(wstart, 2 * _BQ), :]
        vw = v_ref[pl.ds(wstart, 2 * _BQ), :]
        k0 = k_ref[0:_BQ, :]
        v0 = v_ref[0:_BQ, :]
        sw = jax.lax.dot_general(
            q, kw, (((1,), (1,)), ((), ())),
            preferred_element_type=jnp.float32) * _SCALE
        s0 = jax.lax.dot_general(
            q, k0, (((1,), (1,)), ((), ())),
            preferred_element_type=jnp.float32) * _SCALE
        kw_pos = wstart + jax.lax.broadcasted_iota(jnp.int32, (_BQ, 2 * _BQ), 1)
        mask_w = (kw_pos <= qpos) & ((kw_pos < _SINK)
                                     | ((qpos - kw_pos) < _LOCAL))
        k0_pos = jax.lax.broadcasted_iota(jnp.int32, (_BQ, _BQ), 1)
        mask_0 = (k0_pos < _SINK) & (k0_pos < wstart)
        sw = jnp.where(mask_w, sw, _NEG)
        s0 = jnp.where(mask_0, s0, _NEG)
        m = jnp.maximum(jnp.max(sw, axis=1, keepdims=True),
                        jnp.max(s0, axis=1, keepdims=True))
        pw = jnp.exp(sw - m)
        p0 = jnp.exp(s0 - m)
        l = (jnp.sum(pw, axis=1, keepdims=True)
             + jnp.sum(p0, axis=1, keepdims=True))
        o = (jnp.dot(pw.astype(jnp.bfloat16), vw,
                     preferred_element_type=jnp.float32)
             + jnp.dot(p0.astype(jnp.bfloat16), v0,
                       preferred_element_type=jnp.float32))
        o_ref[...] = (o / l).astype(jnp.bfloat16)


def kernel(hidden_states, wqkv, wo):
    hs = hidden_states.reshape(_S, _HID).astype(jnp.bfloat16)
    wq = wqkv.astype(jnp.bfloat16)
    wo_b = wo.astype(jnp.bfloat16)

    half = _HD // 2
    inv_freq = 1.0 / (_ROPE_THETA ** (
        jnp.arange(0, half, dtype=jnp.float32) / half))
    pos = jnp.arange(_S, dtype=jnp.float32)
    angles = pos[:, None] * inv_freq[None, :]
    ctab = jnp.tile(jnp.concatenate([jnp.cos(angles)] * 2, axis=-1),
                    (1, _NQK))
    stab = jnp.tile(jnp.concatenate([jnp.sin(angles)] * 2, axis=-1),
                    (1, _NQK))

    qkv = pl.pallas_call(
        _qkv_rope_kernel,
        grid=(_NQ,),
        in_specs=[
            pl.BlockSpec((_BQ, _HID), lambda i: (i, 0)),
            pl.BlockSpec((_HID, _QKV_W), lambda i: (0, 0)),
            pl.BlockSpec((_BQ, _QK_W), lambda i: (i, 0)),
            pl.BlockSpec((_BQ, _QK_W), lambda i: (i, 0)),
        ],
        out_specs=pl.BlockSpec((_BQ, _QKV_W), lambda i: (i, 0)),
        out_shape=jax.ShapeDtypeStruct((_S, _QKV_W), jnp.bfloat16),
    )(hs, wq, ctab, stab)

    attn = pl.pallas_call(
        _attn_kernel,
        grid=(_NH, _NQ),
        in_specs=[
            pl.BlockSpec((_BQ, _HD), lambda h, qb: (qb, h)),
            pl.BlockSpec((_S, _HD), lambda h, qb: (0, _NH + h // _REP)),
            pl.BlockSpec((_S, _HD),
                         lambda h, qb: (0, _NH + _NKV + h // _REP)),
        ],
        out_specs=pl.BlockSpec((_BQ, _HD), lambda h, qb: (qb, h)),
        out_shape=jax.ShapeDtypeStruct((_S, _NH * _HD), jnp.bfloat16),
    )(qkv, qkv, qkv)

    out = pl.pallas_call(
        _out_proj_kernel,
        grid=(_NQ,),
        in_specs=[
            pl.BlockSpec((_BQ, _NH * _HD), lambda i: (i, 0)),
            pl.BlockSpec((_NH * _HD, _HID), lambda i: (0, 0)),
        ],
        out_specs=pl.BlockSpec((_BQ, _HID), lambda i: (i, 0)),
        out_shape=jax.ShapeDtypeStruct((_S, _HID), jnp.float32),
    )(attn, wo_b)

    return out.reshape(_B, _S, _HID)
